# baseline (speedup 1.0000x reference)
import math

import jax
import jax.numpy as jnp
from jax import lax
from jax.experimental import pallas as pl
from jax.experimental.pallas import tpu as pltpu
from jax.experimental.pallas import tpu_sc as plsc

_N = 99900
_E = 3196800
_NTILES = 32
_LANES = 16
_GRP = 256
_K = 1
_CHUNK = _K * _GRP
_NCHUNK = 396
_RING = 6
_DEPTH = 4
_LAG = _RING - _DEPTH
_EPT_PAD = _NCHUNK * _CHUNK
_ROWTILE = _EPT_PAD // _GRP
_NPAD = 100352
_PROW = _NPAD * _LANES // 128
_PBLK = 128
_GRID = _PROW // _PBLK
_BN_SCALE = 1.0 / math.sqrt(1.0 + 1e-5)


def _fill_rows(buf, nrows, val):
    def body(i, c):
        buf[i, :] = jnp.full((_LANES,), val, jnp.float32)
        return c
    lax.fori_loop(0, nrows, body, 0)


def _zero_my_accum_slice(rows, accum, sid, copyrows):
    lo = sid * copyrows
    n_full = copyrows // _CHUNK
    rem = copyrows % _CHUNK
    for z in range(n_full):
        pltpu.sync_copy(rows, accum.at[pl.ds(lo + z * _CHUNK, _CHUNK)])
    if rem:
        pltpu.sync_copy(rows.at[pl.ds(0, rem)],
                        accum.at[pl.ds(lo + n_full * _CHUNK, rem)])


def _make_sc_pass(npad, rowtile, nchunk, with_gather, interpret=False):
    copyrows = npad // 16
    assert nchunk % _RING == 0

    def body(*refs):
        if with_gather:
            (g_hbm, src_hbm, dst_hbm, out0, out1,
             *bufs) = refs
            srcb = tuple(bufs[0:_RING])
            dstb = tuple(bufs[_RING:2 * _RING])
            rows = tuple(bufs[2 * _RING:3 * _RING])
            accum = bufs[3 * _RING]
            semi = tuple(bufs[3 * _RING + 1:3 * _RING + 1 + _RING])
            semg = tuple(bufs[3 * _RING + 1 + _RING:3 * _RING + 1 + 2 * _RING])
            sems = tuple(bufs[3 * _RING + 1 + 2 * _RING:])
        else:
            (dst_hbm, out0, out1, *bufs) = refs
            dstb = tuple(bufs[0:_RING])
            ones = bufs[_RING]
            rows = (ones,) * _RING
            accum = bufs[_RING + 1]
            semi = tuple(bufs[_RING + 2:_RING + 2 + _RING])
            sems = tuple(bufs[_RING + 2 + _RING:])
        cid = lax.axis_index("c")
        sid = lax.axis_index("s")
        wid = cid * 16 + sid
        _fill_rows(rows[0], _CHUNK, 0.0)
        _zero_my_accum_slice(rows[0], accum, sid, copyrows)
        if not with_gather:
            _fill_rows(rows[0], _CHUNK, 1.0)
        plsc.subcore_barrier()
        base = wid * rowtile

        def stage_idx(m, b, sync):
            r0 = base + m * _K
            if sync:
                if with_gather:
                    pltpu.sync_copy(src_hbm.at[pl.ds(r0, _K)], srcb[b])
                pltpu.sync_copy(dst_hbm.at[pl.ds(r0, _K)], dstb[b])
            else:
                if with_gather:
                    pltpu.async_copy(src_hbm.at[pl.ds(r0, _K)], srcb[b],
                                     semi[b])
                pltpu.async_copy(dst_hbm.at[pl.ds(r0, _K)], dstb[b], semi[b])

        def wait_idx(b):
            if with_gather:
                pltpu.make_async_copy(src_hbm.at[pl.ds(base, _K)], srcb[b],
                                      semi[b]).wait()
            pltpu.make_async_copy(dst_hbm.at[pl.ds(base, _K)], dstb[b],
                                  semi[b]).wait()

        def fire_gathers(b):
            pltpu.async_copy(g_hbm.at[srcb[b].at[0]], rows[b], semg[b])

        def wait_gathers(b):
            pltpu.make_async_copy(g_hbm.at[srcb[b].at[0]], rows[b],
                                  semg[b]).wait()

        def fire_scatters(b):
            pltpu.async_copy(rows[b], accum.at[dstb[b].at[0]], sems[b],
                             add=True)

        def wait_scatters(b):
            pltpu.make_async_copy(rows[b], accum.at[dstb[b].at[0]],
                                  sems[b]).wait()

        for b in range(_DEPTH):
            stage_idx(b, b, sync=True)
            if with_gather:
                fire_gathers(b)

        def ring_iter(p, c):
            for b in range(_RING):
                m = p * _RING + b

                @pl.when(m >= _LAG)
                def _():
                    wait_scatters((b + _RING - _LAG) % _RING)

                @pl.when(m + _DEPTH < nchunk)
                def _():
                    stage_idx(m + _DEPTH, (b + _DEPTH) % _RING, sync=False)
                if with_gather:
                    wait_gathers(b)
                fire_scatters(b)

                @pl.when(m + _DEPTH < nchunk)
                def _():
                    wait_idx((b + _DEPTH) % _RING)
                    if with_gather:
                        fire_gathers((b + _DEPTH) % _RING)
            return c

        lax.fori_loop(0, nchunk // _RING, ring_iter, 0)
        for q in range(_LAG):
            wait_scatters((nchunk - _LAG + q) % _RING)
        plsc.subcore_barrier()
        lo = sid * copyrows

        @pl.when(cid == 0)
        def _():
            pltpu.sync_copy(accum.at[pl.ds(lo, copyrows)],
                            out0.at[pl.ds(lo, copyrows)])

        @pl.when(cid == 1)
        def _():
            pltpu.sync_copy(accum.at[pl.ds(lo, copyrows)],
                            out1.at[pl.ds(lo, copyrows)])

    idxbuf = pltpu.VMEM((_K, _GRP), jnp.int32)
    rowbuf = pltpu.VMEM((_CHUNK, _LANES), jnp.float32)
    dma = pltpu.SemaphoreType.DMA
    part = jax.ShapeDtypeStruct((npad, _LANES), jnp.float32)
    if with_gather:
        scratch = ([idxbuf] * (2 * _RING) + [rowbuf] * _RING +
                   [pltpu.VMEM_SHARED((npad, _LANES), jnp.float32)] +
                   [dma] * (3 * _RING))
    else:
        scratch = ([idxbuf] * _RING + [rowbuf] +
                   [pltpu.VMEM_SHARED((npad, _LANES), jnp.float32)] +
                   [dma] * (2 * _RING))
    return pl.kernel(
        body,
        out_type=(part, part),
        mesh=plsc.VectorSubcoreMesh(core_axis_name="c", subcore_axis_name="s"),
        scratch_types=scratch,
        compiler_params=pltpu.CompilerParams(use_tc_tiling_on_sc=False),
        interpret=interpret,
    )


_sc_agg_raw = _make_sc_pass(_NPAD, _ROWTILE, _NCHUNK, with_gather=True)
_sc_deg_raw = _make_sc_pass(_NPAD, _ROWTILE, _NCHUNK, with_gather=False)


def _sc_agg(gp, srcp, dstp):
    o0, o1 = _sc_agg_raw(gp.reshape(_NPAD, _LANES), srcp, dstp)
    return o0.reshape(_PROW, 128), o1.reshape(_PROW, 128)


def _sc_deg(dstp):
    o0, o1 = _sc_deg_raw(dstp)
    return o0.reshape(_PROW, 128), o1.reshape(_PROW, 128)


def _pspec():
    return pl.BlockSpec((_PBLK, 128), lambda i: (i, 0))


def _full_spec(shape):
    nd = len(shape)
    return pl.BlockSpec(shape, lambda i, _n=nd: (0,) * _n)


def _f0_body(p0_ref, p1_ref, xa_ref, xb_ref,
             w1f_ref, sh1_ref, w2f_ref, sh2_ref, w1_ref,
             dinv_ref, ga_ref, gb_ref):
    dinv = lax.rsqrt(1.0 + p0_ref[...] + p1_ref[...])
    xcat = jnp.concatenate([xa_ref[...], xb_ref[...]], axis=1)
    h1 = jnp.concatenate(
        [jnp.maximum(xcat @ w1f_ref[k] + sh1_ref[k], 0.0) for k in range(7)],
        axis=1)
    h2a = h1 @ w2f_ref[0] + sh2_ref[0]
    h2b = h1 @ w2f_ref[1] + sh2_ref[1]
    hcat = jnp.concatenate([h2a, h2b], axis=1)
    dinv_ref[...] = dinv
    ga_ref[...] = dinv * (hcat @ w1_ref[0])
    gb_ref[...] = dinv * (hcat @ w1_ref[1])


_f0_call = pl.pallas_call(
    _f0_body,
    grid=(_GRID,),
    in_specs=[
        _pspec(), _pspec(), _pspec(), _pspec(),
        _full_spec((7, 256, 128)), _full_spec((7, 1, 128)),
        _full_spec((2, 896, 128)), _full_spec((2, 1, 128)),
        _full_spec((2, 256, 128)),
    ],
    out_specs=[_pspec(), _pspec(), _pspec()],
    out_shape=[jax.ShapeDtypeStruct((_PROW, 128), jnp.float32)] * 3,
)


def _f1_body(sa0, sa1, sb0, sb1, ga_ref, gb_ref, dinv_ref,
             bta_ref, btb_ref, w2_ref, out_ref):
    dinv = dinv_ref[...]
    ha = jnp.maximum(dinv * (sa0[...] + sa1[...] + ga_ref[...])
                     + bta_ref[...], 0.0)
    hb = jnp.maximum(dinv * (sb0[...] + sb1[...] + gb_ref[...])
                     + btb_ref[...], 0.0)
    hcat = jnp.concatenate([ha, hb], axis=1)
    out_ref[...] = dinv * (hcat @ w2_ref[...])


_f1_call = pl.pallas_call(
    _f1_body,
    grid=(_GRID,),
    in_specs=[_pspec()] * 7 + [
        _full_spec((1, 128)), _full_spec((1, 128)), _full_spec((256, 128))],
    out_specs=_pspec(),
    out_shape=jax.ShapeDtypeStruct((_PROW, 128), jnp.float32),
)


def _fmid_body(s0, s1, g_ref, dinv_ref, bt_ref, w_ref, out_ref):
    dinv = dinv_ref[...]
    h = jnp.maximum(dinv * (s0[...] + s1[...] + g_ref[...]) + bt_ref[...],
                    0.0)
    out_ref[...] = dinv * (h @ w_ref[...])


_fmid_call = pl.pallas_call(
    _fmid_body,
    grid=(_GRID,),
    in_specs=[_pspec()] * 4 + [_full_spec((1, 128)), _full_spec((128, 128))],
    out_specs=_pspec(),
    out_shape=jax.ShapeDtypeStruct((_PROW, 128), jnp.float32),
)


def _flast_body(s0, s1, g_ref, dinv_ref, bt_ref, out_ref):
    h = jnp.maximum(
        dinv_ref[...] * (s0[...] + s1[...] + g_ref[...]) + bt_ref[...], 0.0)
    out_ref[...] = h


_flast_call = pl.pallas_call(
    _flast_body,
    grid=(_GRID,),
    in_specs=[_pspec()] * 4 + [_full_spec((1, 128))],
    out_specs=_pspec(),
    out_shape=jax.ShapeDtypeStruct((_PROW, 128), jnp.float32),
)


def _f6_body(h_ref, w_ref, b_ref, out_ref):
    out_ref[...] = h_ref[...] @ w_ref[...] + b_ref[...]


_f6_call = pl.pallas_call(
    _f6_body,
    grid=(1,),
    in_specs=[_full_spec((333, 1200)), _full_spec((1200, 4)),
              _full_spec((1, 4))],
    out_specs=_full_spec((333, 4)),
    out_shape=jax.ShapeDtypeStruct((333, 4), jnp.float32),
)


def _bd(w16):
    return jnp.kron(jnp.eye(8, dtype=jnp.float32), w16)


def _pad2(w, r, c):
    return jnp.pad(w, ((0, r - w.shape[0]), (0, c - w.shape[1])))


def _tile8(v16):
    return jnp.tile(v16, 8).reshape(1, 128)


def kernel(x, edge_index, ffn_w1, ffn_b1, bn1_g, bn1_b, ffn_w2, ffn_b2,
           bn2_g, bn2_b, w1, b1, w2, b2, w3, b3, w4, b4, w5, b5, fc_w, fc_b):
    f32 = jnp.float32
    x = x.astype(f32)
    pad = _NTILES * _EPT_PAD - _E
    srcp = jnp.pad(edge_index[0], (0, pad),
                   constant_values=_N).reshape(-1, _GRP)
    dstp = jnp.pad(edge_index[1], (0, pad),
                   constant_values=_N).reshape(-1, _GRP)
    xq = jnp.pad(x, ((0, _NPAD - _N), (0, 7)))
    xa = xq[:, :16].reshape(_PROW, 128)
    xb = xq[:, 16:].reshape(_PROW, 128)

    s1 = bn1_g * _BN_SCALE
    w1f = ffn_w1 * s1[None, :]
    sh1 = ffn_b1 * s1 + bn1_b
    s2 = bn2_g * _BN_SCALE
    w2f = ffn_w2 * s2[None, :]
    sh2 = ffn_b2 * s2 + bn2_b
    w1fp = _pad2(w1f, 32, 112)
    w2fp = _pad2(w2f, 112, 32)
    w1p = _pad2(w1, 32, 32)
    sh1p = jnp.pad(sh1, (0, 12))
    sh2p = jnp.pad(sh2, (0, 7))
    w1f_bd = jnp.stack([
        jnp.concatenate([_bd(w1fp[0:16, 16 * k:16 * k + 16]),
                         _bd(w1fp[16:32, 16 * k:16 * k + 16])], axis=0)
        for k in range(7)])
    sh1t = jnp.stack([_tile8(sh1p[16 * k:16 * k + 16]) for k in range(7)])
    w2f_bd = jnp.stack([
        jnp.concatenate([_bd(w2fp[16 * k:16 * k + 16, 16 * c:16 * c + 16])
                         for k in range(7)], axis=0)
        for c in range(2)])
    sh2t = jnp.stack([_tile8(sh2p[0:16]), _tile8(sh2p[16:32])])
    w1_bd = jnp.stack([
        jnp.concatenate([_bd(w1p[0:16, 16 * c:16 * c + 16]),
                         _bd(w1p[16:32, 16 * c:16 * c + 16])], axis=0)
        for c in range(2)])
    b1p = jnp.pad(b1, (0, 7))
    bt1a = _tile8(b1p[0:16])
    bt1b = _tile8(b1p[16:32])
    w2p = _pad2(w2, 32, 16)
    w2_st = jnp.concatenate([_bd(w2p[0:16]), _bd(w2p[16:32])], axis=0)
    bd3 = _bd(w3)
    bd4 = _bd(_pad2(w4, 16, 16))
    bd5 = _bd(_pad2(w5, 16, 16))
    bt2 = _tile8(b2)
    bt3 = _tile8(b3)
    bt4 = _tile8(jnp.pad(b4, (0, 8)))
    bt5 = _tile8(jnp.pad(b5, (0, 12)))

    p0, p1 = _sc_deg(dstp)
    dinv, g1a, g1b = _f0_call(p0, p1, xa, xb, w1f_bd, sh1t, w2f_bd, sh2t,
                              w1_bd)
    sa0, sa1 = _sc_agg(g1a, srcp, dstp)
    sb0, sb1 = _sc_agg(g1b, srcp, dstp)
    g2 = _f1_call(sa0, sa1, sb0, sb1, g1a, g1b, dinv, bt1a, bt1b, w2_st)
    s0, s1_ = _sc_agg(g2, srcp, dstp)
    g3 = _fmid_call(s0, s1_, g2, dinv, bt2, bd3)
    s0, s1_ = _sc_agg(g3, srcp, dstp)
    g4 = _fmid_call(s0, s1_, g3, dinv, bt3, bd4)
    s0, s1_ = _sc_agg(g4, srcp, dstp)
    g5 = _fmid_call(s0, s1_, g4, dinv, bt4, bd5)
    s0, s1_ = _sc_agg(g5, srcp, dstp)
    h5 = _flast_call(s0, s1_, g5, dinv, bt5)
    h5 = h5.reshape(_NPAD, 16)[:_N, :4].reshape(_N // 300, 1200)
    return _f6_call(h5, fc_w, fc_b.reshape(1, 4))


# speedup vs baseline: 1.0653x; 1.0653x over previous
import math

import jax
import jax.numpy as jnp
from jax import lax
from jax.experimental import pallas as pl
from jax.experimental.pallas import tpu as pltpu
from jax.experimental.pallas import tpu_sc as plsc

_N = 99900
_E = 3196800
_NTILES = 32
_LANES = 16
_GRP = 512
_K = 1
_CHUNK = _K * _GRP
_NCHUNK = 198
_RING = 3
_DEPTH = 2
_LAG = _RING - _DEPTH
_EPT_PAD = _NCHUNK * _CHUNK
_ROWTILE = _EPT_PAD // _GRP
_NPAD = 100352
_PROW = _NPAD * _LANES // 128
_PBLK = 128
_GRID = _PROW // _PBLK
_BN_SCALE = 1.0 / math.sqrt(1.0 + 1e-5)


def _fill_rows(buf, nrows, val):
    def body(i, c):
        buf[i, :] = jnp.full((_LANES,), val, jnp.float32)
        return c
    lax.fori_loop(0, nrows, body, 0)


def _zero_my_accum_slice(rows, accum, sid, copyrows):
    lo = sid * copyrows
    n_full = copyrows // _CHUNK
    rem = copyrows % _CHUNK
    for z in range(n_full):
        pltpu.sync_copy(rows, accum.at[pl.ds(lo + z * _CHUNK, _CHUNK)])
    if rem:
        pltpu.sync_copy(rows.at[pl.ds(0, rem)],
                        accum.at[pl.ds(lo + n_full * _CHUNK, rem)])


def _make_sc_pass(npad, rowtile, nchunk, with_gather, interpret=False):
    copyrows = npad // 16
    assert nchunk % _RING == 0

    def body(*refs):
        if with_gather:
            (g_hbm, src_hbm, dst_hbm, out0, out1,
             *bufs) = refs
            srcb = tuple(bufs[0:_RING])
            dstb = tuple(bufs[_RING:2 * _RING])
            rows = tuple(bufs[2 * _RING:3 * _RING])
            accum = bufs[3 * _RING]
            semi = tuple(bufs[3 * _RING + 1:3 * _RING + 1 + _RING])
            semg = tuple(bufs[3 * _RING + 1 + _RING:3 * _RING + 1 + 2 * _RING])
            sems = tuple(bufs[3 * _RING + 1 + 2 * _RING:])
        else:
            (dst_hbm, out0, out1, *bufs) = refs
            dstb = tuple(bufs[0:_RING])
            ones = bufs[_RING]
            rows = (ones,) * _RING
            accum = bufs[_RING + 1]
            semi = tuple(bufs[_RING + 2:_RING + 2 + _RING])
            sems = tuple(bufs[_RING + 2 + _RING:])
        cid = lax.axis_index("c")
        sid = lax.axis_index("s")
        wid = cid * 16 + sid
        _fill_rows(rows[0], _CHUNK, 0.0)
        _zero_my_accum_slice(rows[0], accum, sid, copyrows)
        if not with_gather:
            _fill_rows(rows[0], _CHUNK, 1.0)
        plsc.subcore_barrier()
        base = wid * rowtile

        def stage_idx(m, b, sync):
            r0 = base + m * _K
            if sync:
                if with_gather:
                    pltpu.sync_copy(src_hbm.at[pl.ds(r0, _K)], srcb[b])
                pltpu.sync_copy(dst_hbm.at[pl.ds(r0, _K)], dstb[b])
            else:
                if with_gather:
                    pltpu.async_copy(src_hbm.at[pl.ds(r0, _K)], srcb[b],
                                     semi[b])
                pltpu.async_copy(dst_hbm.at[pl.ds(r0, _K)], dstb[b], semi[b])

        def wait_idx(b):
            if with_gather:
                pltpu.make_async_copy(src_hbm.at[pl.ds(base, _K)], srcb[b],
                                      semi[b]).wait()
            pltpu.make_async_copy(dst_hbm.at[pl.ds(base, _K)], dstb[b],
                                  semi[b]).wait()

        def fire_gathers(b):
            pltpu.async_copy(g_hbm.at[srcb[b].at[0]], rows[b], semg[b])

        def wait_gathers(b):
            pltpu.make_async_copy(g_hbm.at[srcb[b].at[0]], rows[b],
                                  semg[b]).wait()

        def fire_scatters(b):
            pltpu.async_copy(rows[b], accum.at[dstb[b].at[0]], sems[b],
                             add=True)

        def wait_scatters(b):
            pltpu.make_async_copy(rows[b], accum.at[dstb[b].at[0]],
                                  sems[b]).wait()

        for b in range(_DEPTH):
            stage_idx(b, b, sync=True)
            if with_gather:
                fire_gathers(b)

        def ring_iter(p, c):
            for b in range(_RING):
                m = p * _RING + b

                @pl.when(m >= _LAG)
                def _():
                    wait_scatters((b + _RING - _LAG) % _RING)

                @pl.when(m + _DEPTH < nchunk)
                def _():
                    stage_idx(m + _DEPTH, (b + _DEPTH) % _RING, sync=False)
                if with_gather:
                    wait_gathers(b)
                fire_scatters(b)

                @pl.when(m + _DEPTH < nchunk)
                def _():
                    wait_idx((b + _DEPTH) % _RING)
                    if with_gather:
                        fire_gathers((b + _DEPTH) % _RING)
            return c

        lax.fori_loop(0, nchunk // _RING, ring_iter, 0)
        for q in range(_LAG):
            wait_scatters((nchunk - _LAG + q) % _RING)
        plsc.subcore_barrier()
        lo = sid * copyrows

        @pl.when(cid == 0)
        def _():
            pltpu.sync_copy(accum.at[pl.ds(lo, copyrows)],
                            out0.at[pl.ds(lo, copyrows)])

        @pl.when(cid == 1)
        def _():
            pltpu.sync_copy(accum.at[pl.ds(lo, copyrows)],
                            out1.at[pl.ds(lo, copyrows)])

    idxbuf = pltpu.VMEM((_K, _GRP), jnp.int32)
    rowbuf = pltpu.VMEM((_CHUNK, _LANES), jnp.float32)
    dma = pltpu.SemaphoreType.DMA
    part = jax.ShapeDtypeStruct((npad, _LANES), jnp.float32)
    if with_gather:
        scratch = ([idxbuf] * (2 * _RING) + [rowbuf] * _RING +
                   [pltpu.VMEM_SHARED((npad, _LANES), jnp.float32)] +
                   [dma] * (3 * _RING))
    else:
        scratch = ([idxbuf] * _RING + [rowbuf] +
                   [pltpu.VMEM_SHARED((npad, _LANES), jnp.float32)] +
                   [dma] * (2 * _RING))
    return pl.kernel(
        body,
        out_type=(part, part),
        mesh=plsc.VectorSubcoreMesh(core_axis_name="c", subcore_axis_name="s"),
        scratch_types=scratch,
        compiler_params=pltpu.CompilerParams(use_tc_tiling_on_sc=False),
        interpret=interpret,
    )


_sc_agg_raw = _make_sc_pass(_NPAD, _ROWTILE, _NCHUNK, with_gather=True)
_sc_deg_raw = _make_sc_pass(_NPAD, _ROWTILE, _NCHUNK, with_gather=False)


def _sc_agg(gp, srcp, dstp):
    o0, o1 = _sc_agg_raw(gp.reshape(_NPAD, _LANES), srcp, dstp)
    return o0.reshape(_PROW, 128), o1.reshape(_PROW, 128)


def _sc_deg(dstp):
    o0, o1 = _sc_deg_raw(dstp)
    return o0.reshape(_PROW, 128), o1.reshape(_PROW, 128)


def _pspec():
    return pl.BlockSpec((_PBLK, 128), lambda i: (i, 0))


def _full_spec(shape):
    nd = len(shape)
    return pl.BlockSpec(shape, lambda i, _n=nd: (0,) * _n)


def _f0_body(p0_ref, p1_ref, xa_ref, xb_ref,
             w1f_ref, sh1_ref, w2f_ref, sh2_ref, w1_ref,
             dinv_ref, ga_ref, gb_ref):
    dinv = lax.rsqrt(1.0 + p0_ref[...] + p1_ref[...])
    xcat = jnp.concatenate([xa_ref[...], xb_ref[...]], axis=1)
    h1 = jnp.concatenate(
        [jnp.maximum(xcat @ w1f_ref[k] + sh1_ref[k], 0.0) for k in range(7)],
        axis=1)
    h2a = h1 @ w2f_ref[0] + sh2_ref[0]
    h2b = h1 @ w2f_ref[1] + sh2_ref[1]
    hcat = jnp.concatenate([h2a, h2b], axis=1)
    dinv_ref[...] = dinv
    ga_ref[...] = dinv * (hcat @ w1_ref[0])
    gb_ref[...] = dinv * (hcat @ w1_ref[1])


_f0_call = pl.pallas_call(
    _f0_body,
    grid=(_GRID,),
    in_specs=[
        _pspec(), _pspec(), _pspec(), _pspec(),
        _full_spec((7, 256, 128)), _full_spec((7, 1, 128)),
        _full_spec((2, 896, 128)), _full_spec((2, 1, 128)),
        _full_spec((2, 256, 128)),
    ],
    out_specs=[_pspec(), _pspec(), _pspec()],
    out_shape=[jax.ShapeDtypeStruct((_PROW, 128), jnp.float32)] * 3,
)


def _f1_body(sa0, sa1, sb0, sb1, ga_ref, gb_ref, dinv_ref,
             bta_ref, btb_ref, w2_ref, out_ref):
    dinv = dinv_ref[...]
    ha = jnp.maximum(dinv * (sa0[...] + sa1[...] + ga_ref[...])
                     + bta_ref[...], 0.0)
    hb = jnp.maximum(dinv * (sb0[...] + sb1[...] + gb_ref[...])
                     + btb_ref[...], 0.0)
    hcat = jnp.concatenate([ha, hb], axis=1)
    out_ref[...] = dinv * (hcat @ w2_ref[...])


_f1_call = pl.pallas_call(
    _f1_body,
    grid=(_GRID,),
    in_specs=[_pspec()] * 7 + [
        _full_spec((1, 128)), _full_spec((1, 128)), _full_spec((256, 128))],
    out_specs=_pspec(),
    out_shape=jax.ShapeDtypeStruct((_PROW, 128), jnp.float32),
)


def _fmid_body(s0, s1, g_ref, dinv_ref, bt_ref, w_ref, out_ref):
    dinv = dinv_ref[...]
    h = jnp.maximum(dinv * (s0[...] + s1[...] + g_ref[...]) + bt_ref[...],
                    0.0)
    out_ref[...] = dinv * (h @ w_ref[...])


_fmid_call = pl.pallas_call(
    _fmid_body,
    grid=(_GRID,),
    in_specs=[_pspec()] * 4 + [_full_spec((1, 128)), _full_spec((128, 128))],
    out_specs=_pspec(),
    out_shape=jax.ShapeDtypeStruct((_PROW, 128), jnp.float32),
)


def _flast_body(s0, s1, g_ref, dinv_ref, bt_ref, out_ref):
    h = jnp.maximum(
        dinv_ref[...] * (s0[...] + s1[...] + g_ref[...]) + bt_ref[...], 0.0)
    out_ref[...] = h


_flast_call = pl.pallas_call(
    _flast_body,
    grid=(_GRID,),
    in_specs=[_pspec()] * 4 + [_full_spec((1, 128))],
    out_specs=_pspec(),
    out_shape=jax.ShapeDtypeStruct((_PROW, 128), jnp.float32),
)


def _f6_body(h_ref, w_ref, b_ref, out_ref):
    out_ref[...] = h_ref[...] @ w_ref[...] + b_ref[...]


_f6_call = pl.pallas_call(
    _f6_body,
    grid=(1,),
    in_specs=[_full_spec((333, 1200)), _full_spec((1200, 4)),
              _full_spec((1, 4))],
    out_specs=_full_spec((333, 4)),
    out_shape=jax.ShapeDtypeStruct((333, 4), jnp.float32),
)


def _bd(w16):
    return jnp.kron(jnp.eye(8, dtype=jnp.float32), w16)


def _pad2(w, r, c):
    return jnp.pad(w, ((0, r - w.shape[0]), (0, c - w.shape[1])))


def _tile8(v16):
    return jnp.tile(v16, 8).reshape(1, 128)


def kernel(x, edge_index, ffn_w1, ffn_b1, bn1_g, bn1_b, ffn_w2, ffn_b2,
           bn2_g, bn2_b, w1, b1, w2, b2, w3, b3, w4, b4, w5, b5, fc_w, fc_b):
    f32 = jnp.float32
    x = x.astype(f32)
    pad = _NTILES * _EPT_PAD - _E
    srcp = jnp.pad(edge_index[0], (0, pad),
                   constant_values=_N).reshape(-1, _GRP)
    dstp = jnp.pad(edge_index[1], (0, pad),
                   constant_values=_N).reshape(-1, _GRP)
    xq = jnp.pad(x, ((0, _NPAD - _N), (0, 7)))
    xa = xq[:, :16].reshape(_PROW, 128)
    xb = xq[:, 16:].reshape(_PROW, 128)

    s1 = bn1_g * _BN_SCALE
    w1f = ffn_w1 * s1[None, :]
    sh1 = ffn_b1 * s1 + bn1_b
    s2 = bn2_g * _BN_SCALE
    w2f = ffn_w2 * s2[None, :]
    sh2 = ffn_b2 * s2 + bn2_b
    w1fp = _pad2(w1f, 32, 112)
    w2fp = _pad2(w2f, 112, 32)
    w1p = _pad2(w1, 32, 32)
    sh1p = jnp.pad(sh1, (0, 12))
    sh2p = jnp.pad(sh2, (0, 7))
    w1f_bd = jnp.stack([
        jnp.concatenate([_bd(w1fp[0:16, 16 * k:16 * k + 16]),
                         _bd(w1fp[16:32, 16 * k:16 * k + 16])], axis=0)
        for k in range(7)])
    sh1t = jnp.stack([_tile8(sh1p[16 * k:16 * k + 16]) for k in range(7)])
    w2f_bd = jnp.stack([
        jnp.concatenate([_bd(w2fp[16 * k:16 * k + 16, 16 * c:16 * c + 16])
                         for k in range(7)], axis=0)
        for c in range(2)])
    sh2t = jnp.stack([_tile8(sh2p[0:16]), _tile8(sh2p[16:32])])
    w1_bd = jnp.stack([
        jnp.concatenate([_bd(w1p[0:16, 16 * c:16 * c + 16]),
                         _bd(w1p[16:32, 16 * c:16 * c + 16])], axis=0)
        for c in range(2)])
    b1p = jnp.pad(b1, (0, 7))
    bt1a = _tile8(b1p[0:16])
    bt1b = _tile8(b1p[16:32])
    w2p = _pad2(w2, 32, 16)
    w2_st = jnp.concatenate([_bd(w2p[0:16]), _bd(w2p[16:32])], axis=0)
    bd3 = _bd(w3)
    bd4 = _bd(_pad2(w4, 16, 16))
    bd5 = _bd(_pad2(w5, 16, 16))
    bt2 = _tile8(b2)
    bt3 = _tile8(b3)
    bt4 = _tile8(jnp.pad(b4, (0, 8)))
    bt5 = _tile8(jnp.pad(b5, (0, 12)))

    p0, p1 = _sc_deg(dstp)
    dinv, g1a, g1b = _f0_call(p0, p1, xa, xb, w1f_bd, sh1t, w2f_bd, sh2t,
                              w1_bd)
    sa0, sa1 = _sc_agg(g1a, srcp, dstp)
    sb0, sb1 = _sc_agg(g1b, srcp, dstp)
    g2 = _f1_call(sa0, sa1, sb0, sb1, g1a, g1b, dinv, bt1a, bt1b, w2_st)
    s0, s1_ = _sc_agg(g2, srcp, dstp)
    g3 = _fmid_call(s0, s1_, g2, dinv, bt2, bd3)
    s0, s1_ = _sc_agg(g3, srcp, dstp)
    g4 = _fmid_call(s0, s1_, g3, dinv, bt3, bd4)
    s0, s1_ = _sc_agg(g4, srcp, dstp)
    g5 = _fmid_call(s0, s1_, g4, dinv, bt4, bd5)
    s0, s1_ = _sc_agg(g5, srcp, dstp)
    h5 = _flast_call(s0, s1_, g5, dinv, bt5)
    h5 = h5.reshape(_NPAD, 16)[:_N, :4].reshape(_N // 300, 1200)
    return _f6_call(h5, fc_w, fc_b.reshape(1, 4))
